# baseline (speedup 1.0000x reference)
import jax
import jax.numpy as jnp
import numpy as np
from jax.experimental import pallas as pl
from jax.experimental.pallas import tpu as pltpu
from jax.experimental.pallas import tpu_sc as plsc

_SIZE = 1600000
_COLS = 128
_ROWS = _SIZE // _COLS

_NC = 2
_NS = 16
_NW = _NC * _NS
_PER_W = _SIZE // _NW
_CHUNK = 10000
_NCHUNK = _PER_W // _CHUNK
_VECS = _CHUNK // 16
_UNROLL = 5

_NBINS = 4096
_HIST_WORDS = _NBINS * 16

_CAP = 4096
_CAPP = _CAP + 16


def _sc_keys_hist_body(probs_hbm, noise_hbm, zeros_hbm, skey_hbm, hist_hbm,
                       pbuf, nbuf, kbuf, hist):
    wid = jax.lax.axis_index("s") * _NC + jax.lax.axis_index("c")
    lane = jax.lax.iota(jnp.int32, 16)
    laneoff = lane * _NBINS + 2048
    ones = (lane >= 0).astype(jnp.int32)

    pltpu.sync_copy(zeros_hbm, hist)

    def chunk(c, _):
        base = wid * _PER_W + c * _CHUNK
        pltpu.sync_copy(probs_hbm.at[pl.ds(base, _CHUNK)], pbuf)
        pltpu.sync_copy(noise_hbm.at[pl.ds(base, _CHUNK)], nbuf)

        def vec(i, _):
            for j in range(_UNROLL):
                sl = pl.ds((i * _UNROLL + j) * 16, 16)
                x = pbuf[sl] + nbuf[sl]
                k = plsc.bitcast(x, jnp.int32)
                s = k ^ (jax.lax.shift_right_arithmetic(k, 31)
                         & jnp.int32(0x7FFFFFFF))
                kbuf[sl] = s
                idx = laneoff + jax.lax.shift_right_arithmetic(s, 20)
                plsc.addupdate_scatter(hist, [idx], ones)
            return 0

        jax.lax.fori_loop(0, _VECS // _UNROLL, vec, 0)
        pltpu.sync_copy(kbuf, skey_hbm.at[pl.ds(base, _CHUNK)])
        return 0

    jax.lax.fori_loop(0, _NCHUNK, chunk, 0)
    pltpu.sync_copy(hist, hist_hbm.at[wid])


def _tc_hist_body(n_ref, hist_ref, scal_ref):
    n = n_ref[0, 0]
    total = jnp.sum(hist_ref[...], axis=0)
    r = jax.lax.broadcasted_iota(jnp.int32, (32, 128), 0)
    c = jax.lax.broadcasted_iota(jnp.int32, (32, 128), 1)
    bin2 = r * 128 + c

    def hstep(b, B):
        cand = B | jnp.left_shift(jnp.int32(1), jnp.int32(11) - b)
        cnt = jnp.sum(jnp.where(bin2 >= cand, total, 0))
        return jnp.where(cnt >= n, cand, B)

    B = jax.lax.fori_loop(0, 12, hstep, jnp.int32(0))
    cnt_above = jnp.sum(jnp.where(bin2 > B, total, 0))
    c8 = jax.lax.broadcasted_iota(jnp.int32, (8, 128), 1)
    scal_ref[...] = jnp.where(c8 < 64, B, cnt_above)


def _sc_compact_body(skey_hbm, scal_hbm, negs_hbm, compact_hbm,
                     kbuf, sbuf, cbuf):
    wid = jax.lax.axis_index("s") * _NC + jax.lax.axis_index("c")
    pltpu.sync_copy(scal_hbm.at[pl.ds(0, 16)], sbuf)
    bvec = sbuf[...]
    pltpu.sync_copy(negs_hbm, cbuf)

    def chunk(c, off):
        base = wid * _PER_W + c * _CHUNK
        pltpu.sync_copy(skey_hbm.at[pl.ds(base, _CHUNK)], kbuf)

        def vec(i, off):
            for j in range(_UNROLL):
                sl = pl.ds((i * _UNROLL + j) * 16, 16)
                s = kbuf[sl]
                m = (jax.lax.shift_right_arithmetic(s, 20) + 2048) == bvec
                low = s & jnp.int32(0xFFFFF)
                safe = jnp.minimum(off, jnp.int32(_CAP))
                plsc.store_compressed(cbuf.at[pl.ds(safe, 16)], low, mask=m)
                off = off + jnp.sum(m.astype(jnp.int32))
            return off

        return jax.lax.fori_loop(0, _VECS // _UNROLL, vec, off)

    jax.lax.fori_loop(0, _NCHUNK, chunk, jnp.int32(0))
    pltpu.sync_copy(cbuf, compact_hbm.at[wid])


def _tc_final_body(n_ref, scal_ref, skey_ref, compact_ref, out_ref):
    n = n_ref[0, 0]
    B = scal_ref[0, 0]
    cnt_above = scal_ref[0, 64]
    v = compact_ref[...]

    def lstep(b, L):
        cand = L | jnp.left_shift(jnp.int32(1), jnp.int32(19) - b)
        cnt = cnt_above + jnp.sum((v >= cand).astype(jnp.int32))
        return jnp.where(cnt >= n, cand, L)

    L = jax.lax.fori_loop(0, 20, lstep, jnp.int32(0))
    thr_s = jnp.left_shift(B ^ jnp.int32(2048), 20) | L
    out_ref[...] = (skey_ref[...] >= thr_s).astype(jnp.float32)


def kernel(mst_probs, n):
    noise = jax.random.gumbel(jax.random.key(42), mst_probs.shape,
                              mst_probs.dtype)
    zeros = jnp.zeros((_HIST_WORDS,), jnp.int32)
    negs = jnp.full((_CAPP,), -1, jnp.int32)
    n_arr = jnp.asarray(n, jnp.int32).reshape(1, 1)

    mesh = plsc.VectorSubcoreMesh(core_axis_name="c", subcore_axis_name="s",
                                  num_cores=_NC, num_subcores=_NS)
    sc_params = pltpu.CompilerParams(needs_layout_passes=False)

    skeys, hists = pl.kernel(
        _sc_keys_hist_body,
        out_type=[
            jax.ShapeDtypeStruct((_SIZE,), jnp.int32),
            jax.ShapeDtypeStruct((_NW, _HIST_WORDS), jnp.int32),
        ],
        mesh=mesh,
        compiler_params=sc_params,
        scratch_types=[
            pltpu.VMEM((_CHUNK,), jnp.float32),
            pltpu.VMEM((_CHUNK,), jnp.float32),
            pltpu.VMEM((_CHUNK,), jnp.int32),
            pltpu.VMEM((_HIST_WORDS,), jnp.int32),
        ],
    )(mst_probs, noise, zeros)

    scal = pl.pallas_call(
        _tc_hist_body,
        out_shape=jax.ShapeDtypeStruct((8, 128), jnp.int32),
        in_specs=[
            pl.BlockSpec(memory_space=pltpu.SMEM),
            pl.BlockSpec(memory_space=pltpu.VMEM),
        ],
        out_specs=pl.BlockSpec(memory_space=pltpu.VMEM),
    )(n_arr, hists.reshape(_NW * 16, 32, 128))

    compact = pl.kernel(
        _sc_compact_body,
        out_type=jax.ShapeDtypeStruct((_NW, _CAPP), jnp.int32),
        mesh=mesh,
        compiler_params=sc_params,
        scratch_types=[
            pltpu.VMEM((_CHUNK,), jnp.int32),
            pltpu.VMEM((16,), jnp.int32),
            pltpu.VMEM((_CAPP,), jnp.int32),
        ],
    )(skeys, scal.reshape(8 * 128), negs)

    out = pl.pallas_call(
        _tc_final_body,
        out_shape=jax.ShapeDtypeStruct((_ROWS, _COLS), jnp.float32),
        in_specs=[
            pl.BlockSpec(memory_space=pltpu.SMEM),
            pl.BlockSpec(memory_space=pltpu.SMEM),
            pl.BlockSpec(memory_space=pltpu.VMEM),
            pl.BlockSpec(memory_space=pltpu.VMEM),
        ],
        out_specs=pl.BlockSpec(memory_space=pltpu.VMEM),
    )(n_arr, scal, skeys.reshape(_ROWS, _COLS),
      compact.reshape(_NW * _CAPP // _COLS, _COLS))
    return out.reshape(_SIZE)


# speedup vs baseline: 1.0310x; 1.0310x over previous
import jax
import jax.numpy as jnp
import numpy as np
from jax.experimental import pallas as pl
from jax.experimental.pallas import tpu as pltpu
from jax.experimental.pallas import tpu_sc as plsc

_SIZE = 1600000
_COLS = 128
_ROWS = _SIZE // _COLS

_NC = 2
_NS = 16
_NW = _NC * _NS
_PER_W = _SIZE // _NW
_CHUNK = 10000
_NCHUNK = _PER_W // _CHUNK
_VECS = _CHUNK // 16
_UNROLL = 5

_NBINS = 65536

_LCAP = 257
_CAPP = 16 * _LCAP + 16


def _sc_keys_hist_body(probs_hbm, noise_hbm, zeros_hbm, skey_hbm, hist_hbm,
                       pbuf, nbuf, kbuf, hist):
    wid = jax.lax.axis_index("s") * _NC + jax.lax.axis_index("c")
    lane = jax.lax.iota(jnp.int32, 16)
    ones = (lane >= 0).astype(jnp.int32)

    pltpu.sync_copy(zeros_hbm, hist)

    def chunk(c, _):
        base = wid * _PER_W + c * _CHUNK
        pltpu.sync_copy(probs_hbm.at[pl.ds(base, _CHUNK)], pbuf)
        pltpu.sync_copy(noise_hbm.at[pl.ds(base, _CHUNK)], nbuf)

        def vec(i, _):
            for j in range(_UNROLL):
                sl = pl.ds((i * _UNROLL + j) * 16, 16)
                x = pbuf[sl] + nbuf[sl]
                k = plsc.bitcast(x, jnp.int32)
                s = k ^ (jax.lax.shift_right_arithmetic(k, 31)
                         & jnp.int32(0x7FFFFFFF))
                kbuf[sl] = s
                idx = jax.lax.shift_right_arithmetic(s, 16) + 32768
                plsc.addupdate_scatter(hist, [idx], ones)
            return 0

        jax.lax.fori_loop(0, _VECS // _UNROLL, vec, 0)
        pltpu.sync_copy(kbuf, skey_hbm.at[pl.ds(base, _CHUNK)])
        return 0

    jax.lax.fori_loop(0, _NCHUNK, chunk, 0)
    pltpu.sync_copy(hist, hist_hbm.at[wid])


def _tc_hist_body(n_ref, hist_ref, scal_ref):
    n = n_ref[0, 0]
    total = jnp.sum(hist_ref[...], axis=0)
    r = jax.lax.broadcasted_iota(jnp.int32, (512, 128), 0)
    c = jax.lax.broadcasted_iota(jnp.int32, (512, 128), 1)
    bin2 = r * 128 + c

    def hstep(b, B):
        cand = B | jnp.left_shift(jnp.int32(1), jnp.int32(15) - b)
        cnt = jnp.sum(jnp.where(bin2 >= cand, total, 0))
        return jnp.where(cnt >= n, cand, B)

    B = jax.lax.fori_loop(0, 16, hstep, jnp.int32(0))
    cnt_above = jnp.sum(jnp.where(bin2 > B, total, 0))
    c8 = jax.lax.broadcasted_iota(jnp.int32, (8, 128), 1)
    scal_ref[...] = jnp.where(c8 < 64, B, cnt_above)


def _sc_compact_body(skey_hbm, scal_hbm, negs_hbm, compact_hbm,
                     kbuf, sbuf, cbuf):
    wid = jax.lax.axis_index("s") * _NC + jax.lax.axis_index("c")
    lane = jax.lax.iota(jnp.int32, 16)
    lanebase = lane * _LCAP
    pltpu.sync_copy(scal_hbm.at[pl.ds(0, 16)], sbuf)
    bvec = sbuf[...]
    pltpu.sync_copy(negs_hbm, cbuf)

    def chunk(c, cnt):
        base = wid * _PER_W + c * _CHUNK
        pltpu.sync_copy(skey_hbm.at[pl.ds(base, _CHUNK)], kbuf)

        def vec(i, cnt):
            for j in range(_UNROLL):
                sl = pl.ds((i * _UNROLL + j) * 16, 16)
                s = kbuf[sl]
                m = (jax.lax.shift_right_arithmetic(s, 16) + 32768) == bvec
                low = s & jnp.int32(0xFFFF)
                idx = lanebase + jnp.minimum(cnt, _LCAP - 1)
                plsc.store_scatter(cbuf, [idx], low, mask=m)
                cnt = cnt + m.astype(jnp.int32)
            return cnt

        return jax.lax.fori_loop(0, _VECS // _UNROLL, vec, cnt)

    jax.lax.fori_loop(0, _NCHUNK, chunk, jnp.zeros((16,), jnp.int32))
    pltpu.sync_copy(cbuf, compact_hbm.at[wid])


def _tc_final_body(n_ref, scal_ref, skey_ref, compact_ref, out_ref):
    n = n_ref[0, 0]
    B = scal_ref[0, 0]
    cnt_above = scal_ref[0, 64]
    v = compact_ref[...]

    def lstep(b, L):
        cand = L | jnp.left_shift(jnp.int32(1), jnp.int32(15) - b)
        cnt = cnt_above + jnp.sum((v >= cand).astype(jnp.int32))
        return jnp.where(cnt >= n, cand, L)

    L = jax.lax.fori_loop(0, 16, lstep, jnp.int32(0))
    thr_s = jnp.left_shift(B ^ jnp.int32(32768), 16) | L
    out_ref[...] = (skey_ref[...] >= thr_s).astype(jnp.float32)


def kernel(mst_probs, n):
    noise = jax.random.gumbel(jax.random.key(42), mst_probs.shape,
                              mst_probs.dtype)
    zeros = jnp.zeros((_NBINS,), jnp.int32)
    negs = jnp.full((_CAPP,), -1, jnp.int32)
    n_arr = jnp.asarray(n, jnp.int32).reshape(1, 1)

    mesh = plsc.VectorSubcoreMesh(core_axis_name="c", subcore_axis_name="s",
                                  num_cores=_NC, num_subcores=_NS)
    sc_params = pltpu.CompilerParams(needs_layout_passes=False)

    skeys, hists = pl.kernel(
        _sc_keys_hist_body,
        out_type=[
            jax.ShapeDtypeStruct((_SIZE,), jnp.int32),
            jax.ShapeDtypeStruct((_NW, _NBINS), jnp.int32),
        ],
        mesh=mesh,
        compiler_params=sc_params,
        scratch_types=[
            pltpu.VMEM((_CHUNK,), jnp.float32),
            pltpu.VMEM((_CHUNK,), jnp.float32),
            pltpu.VMEM((_CHUNK,), jnp.int32),
            pltpu.VMEM((_NBINS,), jnp.int32),
        ],
    )(mst_probs, noise, zeros)

    scal = pl.pallas_call(
        _tc_hist_body,
        out_shape=jax.ShapeDtypeStruct((8, 128), jnp.int32),
        in_specs=[
            pl.BlockSpec(memory_space=pltpu.SMEM),
            pl.BlockSpec(memory_space=pltpu.VMEM),
        ],
        out_specs=pl.BlockSpec(memory_space=pltpu.VMEM),
    )(n_arr, hists.reshape(_NW, 512, 128))

    compact = pl.kernel(
        _sc_compact_body,
        out_type=jax.ShapeDtypeStruct((_NW, _CAPP), jnp.int32),
        mesh=mesh,
        compiler_params=sc_params,
        scratch_types=[
            pltpu.VMEM((_CHUNK,), jnp.int32),
            pltpu.VMEM((16,), jnp.int32),
            pltpu.VMEM((_CAPP,), jnp.int32),
        ],
    )(skeys, scal.reshape(8 * 128), negs)

    out = pl.pallas_call(
        _tc_final_body,
        out_shape=jax.ShapeDtypeStruct((_ROWS, _COLS), jnp.float32),
        in_specs=[
            pl.BlockSpec(memory_space=pltpu.SMEM),
            pl.BlockSpec(memory_space=pltpu.SMEM),
            pl.BlockSpec(memory_space=pltpu.VMEM),
            pl.BlockSpec(memory_space=pltpu.VMEM),
        ],
        out_specs=pl.BlockSpec(memory_space=pltpu.VMEM),
    )(n_arr, scal, skeys.reshape(_ROWS, _COLS),
      compact.reshape(_NW * _CAPP // _COLS, _COLS))
    return out.reshape(_SIZE)


# speedup vs baseline: 1.1437x; 1.1093x over previous
import jax
import jax.numpy as jnp
import numpy as np
from jax.experimental import pallas as pl
from jax.experimental.pallas import tpu as pltpu
from jax.experimental.pallas import tpu_sc as plsc

_SIZE = 1600000
_COLS = 128
_ROWS = _SIZE // _COLS

_NC = 2
_NS = 16
_NW = _NC * _NS
_PER_W = _SIZE // _NW
_CHUNK = 10000
_NCHUNK = _PER_W // _CHUNK
_VECS = _CHUNK // 16
_UNROLL = 5

_NBINS = 65536

_LCAP = 257
_CAPP = 16 * _LCAP + 16


def _sc_keys_hist_body(probs_hbm, noise_hbm, zeros_hbm, skey_hbm, hist_hbm,
                       pbuf0, nbuf0, kbuf0, pbuf1, nbuf1, kbuf1, hist,
                       sp0, sn0, sk0, sp1, sn1, sk1, sz):
    wid = jax.lax.axis_index("s") * _NC + jax.lax.axis_index("c")
    lane = jax.lax.iota(jnp.int32, 16)
    ones = (lane >= 0).astype(jnp.int32)
    pb, nb, kb = [pbuf0, pbuf1], [nbuf0, nbuf1], [kbuf0, kbuf1]
    sp, sn, sk = [sp0, sp1], [sn0, sn1], [sk0, sk1]

    def base(c):
        return wid * _PER_W + c * _CHUNK

    hz = pltpu.async_copy(zeros_hbm, hist, sz)
    in_h = {0: (pltpu.async_copy(probs_hbm.at[pl.ds(base(0), _CHUNK)],
                                 pb[0], sp[0]),
                pltpu.async_copy(noise_hbm.at[pl.ds(base(0), _CHUNK)],
                                 nb[0], sn[0]))}
    hz.wait()
    out_h = {}
    for c in range(_NCHUNK):
        cur = c & 1
        if c + 1 < _NCHUNK:
            nxt = (c + 1) & 1
            in_h[c + 1] = (
                pltpu.async_copy(probs_hbm.at[pl.ds(base(c + 1), _CHUNK)],
                                 pb[nxt], sp[nxt]),
                pltpu.async_copy(noise_hbm.at[pl.ds(base(c + 1), _CHUNK)],
                                 nb[nxt], sn[nxt]))
        in_h[c][0].wait()
        in_h[c][1].wait()
        if c >= 2:
            out_h[c - 2].wait()
        pbuf, nbuf, kbuf = pb[cur], nb[cur], kb[cur]

        def vec(i, _):
            for j in range(_UNROLL):
                sl = pl.ds((i * _UNROLL + j) * 16, 16)
                x = pbuf[sl] + nbuf[sl]
                k = plsc.bitcast(x, jnp.int32)
                s = k ^ (jax.lax.shift_right_arithmetic(k, 31)
                         & jnp.int32(0x7FFFFFFF))
                kbuf[sl] = s
                idx = jax.lax.shift_right_arithmetic(s, 16) + 32768
                plsc.addupdate_scatter(hist, [idx], ones)
            return 0

        jax.lax.fori_loop(0, _VECS // _UNROLL, vec, 0)
        out_h[c] = pltpu.async_copy(kbuf, skey_hbm.at[pl.ds(base(c), _CHUNK)],
                                    sk[cur])
    out_h[_NCHUNK - 2].wait()
    out_h[_NCHUNK - 1].wait()
    pltpu.sync_copy(hist, hist_hbm.at[wid])


def _tc_hist_body(n_ref, hist_ref, scal_ref):
    n = n_ref[0, 0]
    total = jnp.sum(hist_ref[...], axis=0)
    r = jax.lax.broadcasted_iota(jnp.int32, (512, 128), 0)
    c = jax.lax.broadcasted_iota(jnp.int32, (512, 128), 1)
    bin2 = r * 128 + c

    def hstep(b, B):
        cand = B | jnp.left_shift(jnp.int32(1), jnp.int32(15) - b)
        cnt = jnp.sum(jnp.where(bin2 >= cand, total, 0))
        return jnp.where(cnt >= n, cand, B)

    B = jax.lax.fori_loop(0, 16, hstep, jnp.int32(0))
    cnt_above = jnp.sum(jnp.where(bin2 > B, total, 0))
    c8 = jax.lax.broadcasted_iota(jnp.int32, (8, 128), 1)
    scal_ref[...] = jnp.where(c8 < 64, B, cnt_above)


def _sc_compact_body(skey_hbm, scal_hbm, negs_hbm, compact_hbm,
                     kbuf0, kbuf1, sbuf, cbuf, sa0, sa1, sb, sc_):
    wid = jax.lax.axis_index("s") * _NC + jax.lax.axis_index("c")
    lane = jax.lax.iota(jnp.int32, 16)
    lanebase = lane * _LCAP
    kb, sa = [kbuf0, kbuf1], [sa0, sa1]

    def base(c):
        return wid * _PER_W + c * _CHUNK

    hn = pltpu.async_copy(negs_hbm, cbuf, sc_)
    hs = pltpu.async_copy(scal_hbm.at[pl.ds(0, 16)], sbuf, sb)
    in_h = {0: pltpu.async_copy(skey_hbm.at[pl.ds(base(0), _CHUNK)],
                                kb[0], sa[0])}
    hs.wait()
    bvec = sbuf[...]
    hn.wait()

    cnt = jnp.zeros((16,), jnp.int32)
    for c in range(_NCHUNK):
        cur = c & 1
        if c + 1 < _NCHUNK:
            nxt = (c + 1) & 1
            in_h[c + 1] = pltpu.async_copy(
                skey_hbm.at[pl.ds(base(c + 1), _CHUNK)], kb[nxt], sa[nxt])
        in_h[c].wait()
        kbuf = kb[cur]

        def vec(i, cnt):
            for j in range(_UNROLL):
                sl = pl.ds((i * _UNROLL + j) * 16, 16)
                s = kbuf[sl]
                m = (jax.lax.shift_right_arithmetic(s, 16) + 32768) == bvec
                low = s & jnp.int32(0xFFFF)
                idx = lanebase + jnp.minimum(cnt, _LCAP - 1)
                plsc.store_scatter(cbuf, [idx], low, mask=m)
                cnt = cnt + m.astype(jnp.int32)
            return cnt

        cnt = jax.lax.fori_loop(0, _VECS // _UNROLL, vec, cnt)
    pltpu.sync_copy(cbuf, compact_hbm.at[wid])


def _tc_final_body(n_ref, scal_ref, skey_ref, compact_ref, out_ref):
    n = n_ref[0, 0]
    B = scal_ref[0, 0]
    cnt_above = scal_ref[0, 64]
    v = compact_ref[...]

    def lstep(b, L):
        cand = L | jnp.left_shift(jnp.int32(1), jnp.int32(15) - b)
        cnt = cnt_above + jnp.sum((v >= cand).astype(jnp.int32))
        return jnp.where(cnt >= n, cand, L)

    L = jax.lax.fori_loop(0, 16, lstep, jnp.int32(0))
    thr_s = jnp.left_shift(B ^ jnp.int32(32768), 16) | L
    out_ref[...] = (skey_ref[...] >= thr_s).astype(jnp.float32)


def kernel(mst_probs, n):
    noise = jax.random.gumbel(jax.random.key(42), mst_probs.shape,
                              mst_probs.dtype)
    zeros = jnp.zeros((_NBINS,), jnp.int32)
    negs = jnp.full((_CAPP,), -1, jnp.int32)
    n_arr = jnp.asarray(n, jnp.int32).reshape(1, 1)

    mesh = plsc.VectorSubcoreMesh(core_axis_name="c", subcore_axis_name="s",
                                  num_cores=_NC, num_subcores=_NS)
    sc_params = pltpu.CompilerParams(needs_layout_passes=False)

    skeys, hists = pl.kernel(
        _sc_keys_hist_body,
        out_type=[
            jax.ShapeDtypeStruct((_SIZE,), jnp.int32),
            jax.ShapeDtypeStruct((_NW, _NBINS), jnp.int32),
        ],
        mesh=mesh,
        compiler_params=sc_params,
        scratch_types=[
            pltpu.VMEM((_CHUNK,), jnp.float32),
            pltpu.VMEM((_CHUNK,), jnp.float32),
            pltpu.VMEM((_CHUNK,), jnp.int32),
            pltpu.VMEM((_CHUNK,), jnp.float32),
            pltpu.VMEM((_CHUNK,), jnp.float32),
            pltpu.VMEM((_CHUNK,), jnp.int32),
            pltpu.VMEM((_NBINS,), jnp.int32),
        ] + [pltpu.SemaphoreType.DMA] * 7,
    )(mst_probs, noise, zeros)

    scal = pl.pallas_call(
        _tc_hist_body,
        out_shape=jax.ShapeDtypeStruct((8, 128), jnp.int32),
        in_specs=[
            pl.BlockSpec(memory_space=pltpu.SMEM),
            pl.BlockSpec(memory_space=pltpu.VMEM),
        ],
        out_specs=pl.BlockSpec(memory_space=pltpu.VMEM),
    )(n_arr, hists.reshape(_NW, 512, 128))

    compact = pl.kernel(
        _sc_compact_body,
        out_type=jax.ShapeDtypeStruct((_NW, _CAPP), jnp.int32),
        mesh=mesh,
        compiler_params=sc_params,
        scratch_types=[
            pltpu.VMEM((_CHUNK,), jnp.int32),
            pltpu.VMEM((_CHUNK,), jnp.int32),
            pltpu.VMEM((16,), jnp.int32),
            pltpu.VMEM((_CAPP,), jnp.int32),
        ] + [pltpu.SemaphoreType.DMA] * 4,
    )(skeys, scal.reshape(8 * 128), negs)

    out = pl.pallas_call(
        _tc_final_body,
        out_shape=jax.ShapeDtypeStruct((_ROWS, _COLS), jnp.float32),
        in_specs=[
            pl.BlockSpec(memory_space=pltpu.SMEM),
            pl.BlockSpec(memory_space=pltpu.SMEM),
            pl.BlockSpec(memory_space=pltpu.VMEM),
            pl.BlockSpec(memory_space=pltpu.VMEM),
        ],
        out_specs=pl.BlockSpec(memory_space=pltpu.VMEM),
    )(n_arr, scal, skeys.reshape(_ROWS, _COLS),
      compact.reshape(_NW * _CAPP // _COLS, _COLS))
    return out.reshape(_SIZE)
